# baseline (speedup 1.0000x reference)
import functools

import jax
import jax.numpy as jnp
from jax import lax
from jax.experimental import pallas as pl
from jax.experimental.pallas import tpu as pltpu
from jax.experimental.pallas import tpu_sc as plsc

_NC = 2
_NS = 16
_NW = _NC * _NS
_LANES = 16

_TC_TILE = 512
_NCHUNKS = 4


def _argmax_body(tr_ref, rep_ref, keys_ref, idx_ref, trt_ref, rept_ref):
    tr = tr_ref[...]
    rep = rep_ref[...]
    keys = keys_ref[...]
    s = lax.dot_general(tr, keys, (((1,), (1,)), ((), ())),
                        preferred_element_type=jnp.float32)
    m = jnp.max(s, axis=1, keepdims=True)
    ii = lax.broadcasted_iota(jnp.int32, s.shape, 1)
    cand = jnp.where(s == m, ii, s.shape[1])
    idx_ref[0, 0, :] = jnp.min(cand, axis=1)
    trt_ref[...] = tr.T
    rept_ref[...] = rep.T


def _tc_argmax_t(tr2, rep2, keys, chunk_off, chunk_t):
    t, c = tr2.shape
    m = keys.shape[0]
    grid = chunk_t // _TC_TILE
    base = chunk_off // _TC_TILE
    idx3, trt, rept = pl.pallas_call(
        _argmax_body,
        grid=(grid,),
        in_specs=[
            pl.BlockSpec((_TC_TILE, c), lambda i: (i + base, 0)),
            pl.BlockSpec((_TC_TILE, c), lambda i: (i + base, 0)),
            pl.BlockSpec((m, c), lambda i: (0, 0)),
        ],
        out_specs=[
            pl.BlockSpec((1, 1, _TC_TILE), lambda i: (i, 0, 0)),
            pl.BlockSpec((c, _TC_TILE), lambda i: (0, i)),
            pl.BlockSpec((c, _TC_TILE), lambda i: (0, i)),
        ],
        out_shape=[
            jax.ShapeDtypeStruct((grid, 1, _TC_TILE), jnp.int32),
            jax.ShapeDtypeStruct((c, chunk_t), jnp.float32),
            jax.ShapeDtypeStruct((c, chunk_t), jnp.float32),
        ],
        compiler_params=pltpu.CompilerParams(
            dimension_semantics=("arbitrary",)),
    )(tr2, rep2, keys)
    return idx3.reshape(chunk_t), trt, rept


def _make_sc_mse(chunk_t, c, m):
    per_w = chunk_t // _NW
    n_groups = per_w // _LANES
    mesh = plsc.VectorSubcoreMesh(core_axis_name="c", subcore_axis_name="s",
                                  num_cores=_NC, num_subcores=_NS)

    @functools.partial(
        pl.kernel,
        out_type=[jax.ShapeDtypeStruct((chunk_t,), jnp.float32),
                  jax.ShapeDtypeStruct((chunk_t,), jnp.float32)],
        mesh=mesh,
        scratch_types=[
            pltpu.VMEM((c, m), jnp.float32),
            pltpu.VMEM((c, m), jnp.float32),
            pltpu.VMEM((c, per_w), jnp.float32),
            pltpu.VMEM((c, per_w), jnp.float32),
            pltpu.VMEM((per_w,), jnp.int32),
            pltpu.VMEM((per_w,), jnp.float32),
            pltpu.VMEM((per_w,), jnp.float32),
        ],
        compiler_params=pltpu.CompilerParams(needs_layout_passes=False),
    )
    def sc_mse(tr_hbm, rep_hbm, keys_hbm, values_hbm, idx_hbm,
               outk_hbm, outv_hbm,
               keys_v, values_v, tr_v, rep_v, idx_v, outk_v, outv_v):
        wid = lax.axis_index("s") * _NC + lax.axis_index("c")
        base = wid * per_w
        pltpu.sync_copy(keys_hbm, keys_v)
        pltpu.sync_copy(values_hbm, values_v)
        pltpu.sync_copy(tr_hbm.at[:, pl.ds(base, per_w)], tr_v)
        pltpu.sync_copy(rep_hbm.at[:, pl.ds(base, per_w)], rep_v)
        pltpu.sync_copy(idx_hbm.at[pl.ds(base, per_w)], idx_v)
        lanes = lax.iota(jnp.int32, _LANES)

        def group_body(g, carry):
            rows = g * _LANES + lanes
            idxv = plsc.load_gather(idx_v, [rows])
            acck = jnp.zeros((_LANES,), jnp.float32)
            accv = jnp.zeros((_LANES,), jnp.float32)
            for cc in range(c):
                col = jnp.full((_LANES,), cc, jnp.int32)
                trc = plsc.load_gather(tr_v, [col, rows])
                kc = plsc.load_gather(keys_v, [col, idxv])
                dk = trc - kc
                acck = acck + dk * dk
                rc = plsc.load_gather(rep_v, [col, rows])
                vc = plsc.load_gather(values_v, [col, idxv])
                dv = rc - vc
                accv = accv + dv * dv
            plsc.store_scatter(outk_v, [rows], acck)
            plsc.store_scatter(outv_v, [rows], accv)
            return carry

        lax.fori_loop(0, n_groups, group_body, 0)
        pltpu.sync_copy(outk_v, outk_hbm.at[pl.ds(base, per_w)])
        pltpu.sync_copy(outv_v, outv_hbm.at[pl.ds(base, per_w)])

    return sc_mse


def kernel(trend_representation, representation, keys, values):
    b, l, c = trend_representation.shape
    m = keys.shape[0]
    t = b * l
    tr2 = trend_representation.reshape(t, c)
    rep2 = representation.reshape(t, c)
    keys_t = keys.T
    values_t = values.T
    chunk_t = t // _NCHUNKS
    sc_mse = _make_sc_mse(chunk_t, c, m)
    kgs, vgs = [], []
    for i in range(_NCHUNKS):
        idx, trt, rept = _tc_argmax_t(tr2, rep2, keys, i * chunk_t, chunk_t)
        kg, vg = sc_mse(trt, rept, keys_t, values_t, idx)
        kgs.append(kg)
        vgs.append(vg)
    kg = jnp.concatenate(kgs).reshape(b, l)
    vg = jnp.concatenate(vgs).reshape(b, l)
    return kg, vg


# speedup vs baseline: 1.2052x; 1.2052x over previous
import functools

import jax
import jax.numpy as jnp
from jax import lax
from jax.experimental import pallas as pl
from jax.experimental.pallas import tpu as pltpu
from jax.experimental.pallas import tpu_sc as plsc

_NC = 2
_NS = 16
_NW = _NC * _NS
_LANES = 16

_TC_TILE = 512
_NCHUNKS = 2


def _argmax_body(tr_ref, rep_ref, keys_ref, idx_ref, trt_ref, rept_ref):
    tr = tr_ref[0]
    rep = rep_ref[0]
    keys = keys_ref[...]
    s = lax.dot_general(tr, keys, (((1,), (1,)), ((), ())),
                        preferred_element_type=jnp.float32)
    idx_ref[0, 0, :] = jnp.argmax(s, axis=1).astype(jnp.int32)
    trt_ref[...] = tr.T
    rept_ref[...] = rep.T


def _tc_argmax_t(tr3, rep3, keys, row_off, n_rows):
    b, l, c = tr3.shape
    m = keys.shape[0]
    jpr = l // _TC_TILE
    chunk_t = n_rows * l
    idx3, trt, rept = pl.pallas_call(
        _argmax_body,
        grid=(n_rows, jpr),
        in_specs=[
            pl.BlockSpec((1, _TC_TILE, c), lambda bb, j: (bb + row_off, j, 0)),
            pl.BlockSpec((1, _TC_TILE, c), lambda bb, j: (bb + row_off, j, 0)),
            pl.BlockSpec((m, c), lambda bb, j: (0, 0)),
        ],
        out_specs=[
            pl.BlockSpec((1, 1, _TC_TILE), lambda bb, j: (bb * jpr + j, 0, 0)),
            pl.BlockSpec((c, _TC_TILE), lambda bb, j: (0, bb * jpr + j)),
            pl.BlockSpec((c, _TC_TILE), lambda bb, j: (0, bb * jpr + j)),
        ],
        out_shape=[
            jax.ShapeDtypeStruct((n_rows * jpr, 1, _TC_TILE), jnp.int32),
            jax.ShapeDtypeStruct((c, chunk_t), jnp.float32),
            jax.ShapeDtypeStruct((c, chunk_t), jnp.float32),
        ],
        compiler_params=pltpu.CompilerParams(
            dimension_semantics=("arbitrary", "arbitrary")),
    )(tr3, rep3, keys)
    return idx3.reshape(chunk_t), trt, rept


def _make_sc_mse(chunk_t, c, m):
    per_w = chunk_t // _NW
    n_groups = per_w // _LANES
    mesh = plsc.VectorSubcoreMesh(core_axis_name="c", subcore_axis_name="s",
                                  num_cores=_NC, num_subcores=_NS)

    @functools.partial(
        pl.kernel,
        out_type=[jax.ShapeDtypeStruct((chunk_t,), jnp.float32),
                  jax.ShapeDtypeStruct((chunk_t,), jnp.float32)],
        mesh=mesh,
        scratch_types=[
            pltpu.VMEM((c, m), jnp.float32),
            pltpu.VMEM((c, m), jnp.float32),
            pltpu.VMEM((c, per_w), jnp.float32),
            pltpu.VMEM((c, per_w), jnp.float32),
            pltpu.VMEM((per_w,), jnp.int32),
            pltpu.VMEM((per_w,), jnp.float32),
            pltpu.VMEM((per_w,), jnp.float32),
        ],
        compiler_params=pltpu.CompilerParams(needs_layout_passes=False),
    )
    def sc_mse(tr_hbm, rep_hbm, keys_hbm, values_hbm, idx_hbm,
               outk_hbm, outv_hbm,
               keys_v, values_v, tr_v, rep_v, idx_v, outk_v, outv_v):
        wid = lax.axis_index("s") * _NC + lax.axis_index("c")
        base = wid * per_w
        pltpu.sync_copy(keys_hbm, keys_v)
        pltpu.sync_copy(values_hbm, values_v)
        pltpu.sync_copy(tr_hbm.at[:, pl.ds(base, per_w)], tr_v)
        pltpu.sync_copy(rep_hbm.at[:, pl.ds(base, per_w)], rep_v)
        pltpu.sync_copy(idx_hbm.at[pl.ds(base, per_w)], idx_v)
        lanes = lax.iota(jnp.int32, _LANES)

        def group_body(g, carry):
            rows = g * _LANES + lanes
            idxv = plsc.load_gather(idx_v, [rows])
            acck = jnp.zeros((_LANES,), jnp.float32)
            accv = jnp.zeros((_LANES,), jnp.float32)
            for cc in range(c):
                col = jnp.full((_LANES,), cc, jnp.int32)
                trc = plsc.load_gather(tr_v, [col, rows])
                kc = plsc.load_gather(keys_v, [col, idxv])
                dk = trc - kc
                acck = acck + dk * dk
                rc = plsc.load_gather(rep_v, [col, rows])
                vc = plsc.load_gather(values_v, [col, idxv])
                dv = rc - vc
                accv = accv + dv * dv
            plsc.store_scatter(outk_v, [rows], acck)
            plsc.store_scatter(outv_v, [rows], accv)
            return carry

        lax.fori_loop(0, n_groups, group_body, 0)
        pltpu.sync_copy(outk_v, outk_hbm.at[pl.ds(base, per_w)])
        pltpu.sync_copy(outv_v, outv_hbm.at[pl.ds(base, per_w)])

    return sc_mse


def kernel(trend_representation, representation, keys, values):
    b, l, c = trend_representation.shape
    m = keys.shape[0]
    t = b * l
    keys_t = keys.T
    values_t = values.T
    rows_per_chunk = b // _NCHUNKS
    chunk_t = rows_per_chunk * l
    sc_mse = _make_sc_mse(chunk_t, c, m)
    kgs, vgs = [], []
    for i in range(_NCHUNKS):
        idx, trt, rept = _tc_argmax_t(trend_representation, representation,
                                      keys, i * rows_per_chunk, rows_per_chunk)
        kg, vg = sc_mse(trt, rept, keys_t, values_t, idx)
        kgs.append(kg)
        vgs.append(vg)
    kg = jnp.concatenate(kgs).reshape(b, l)
    vg = jnp.concatenate(vgs).reshape(b, l)
    return kg, vg


# speedup vs baseline: 1.4494x; 1.2027x over previous
import functools

import jax
import jax.numpy as jnp
from jax import lax
from jax.experimental import pallas as pl
from jax.experimental.pallas import tpu as pltpu
from jax.experimental.pallas import tpu_sc as plsc

_NC = 2
_NS = 16
_NW = _NC * _NS
_LANES = 16

_TC_TILE = 512
_NCHUNKS = 2


def _argmax_body(trt_ref, keyst_ref, idx_ref):
    tr_cm = trt_ref[0]
    keys_cm = keyst_ref[...]
    s = lax.dot_general(tr_cm, keys_cm, (((0,), (0,)), ((), ())),
                        preferred_element_type=jnp.float32)
    idx_ref[0, 0, :] = jnp.argmax(s, axis=1).astype(jnp.int32)


def _tc_argmax(trt3, keys_t, row_off, n_rows):
    b, c, l = trt3.shape
    m = keys_t.shape[1]
    jpr = l // _TC_TILE
    chunk_t = n_rows * l
    idx3 = pl.pallas_call(
        _argmax_body,
        grid=(n_rows, jpr),
        in_specs=[
            pl.BlockSpec((1, c, _TC_TILE), lambda bb, j: (bb + row_off, 0, j)),
            pl.BlockSpec((c, m), lambda bb, j: (0, 0)),
        ],
        out_specs=pl.BlockSpec((1, 1, _TC_TILE),
                               lambda bb, j: (bb * jpr + j, 0, 0)),
        out_shape=jax.ShapeDtypeStruct((n_rows * jpr, 1, _TC_TILE), jnp.int32),
        compiler_params=pltpu.CompilerParams(
            dimension_semantics=("arbitrary", "arbitrary")),
    )(trt3, keys_t)
    return idx3.reshape(chunk_t)


def _make_sc_mse(b, l, c, m, chunk_t):
    per_w = chunk_t // _NW
    n_groups = per_w // _LANES
    wpr = l // per_w
    mesh = plsc.VectorSubcoreMesh(core_axis_name="c", subcore_axis_name="s",
                                  num_cores=_NC, num_subcores=_NS)

    def make(row_off):
        @functools.partial(
            pl.kernel,
            out_type=[jax.ShapeDtypeStruct((chunk_t,), jnp.float32),
                      jax.ShapeDtypeStruct((chunk_t,), jnp.float32)],
            mesh=mesh,
            scratch_types=[
                pltpu.VMEM((c, m), jnp.float32),
                pltpu.VMEM((c, m), jnp.float32),
                pltpu.VMEM((c, per_w), jnp.float32),
                pltpu.VMEM((c, per_w), jnp.float32),
                pltpu.VMEM((per_w,), jnp.int32),
                pltpu.VMEM((per_w,), jnp.float32),
                pltpu.VMEM((per_w,), jnp.float32),
            ],
            compiler_params=pltpu.CompilerParams(needs_layout_passes=False),
        )
        def sc_mse(trt_hbm, rept_hbm, keyst_hbm, valuest_hbm, idx_hbm,
                   outk_hbm, outv_hbm,
                   keys_v, values_v, tr_v, rep_v, idx_v, outk_v, outv_v):
            wid = lax.axis_index("s") * _NC + lax.axis_index("c")
            base = wid * per_w
            bb = row_off + wid // wpr
            l0 = (wid % wpr) * per_w
            pltpu.sync_copy(keyst_hbm, keys_v)
            pltpu.sync_copy(valuest_hbm, values_v)
            pltpu.sync_copy(trt_hbm.at[bb, :, pl.ds(l0, per_w)], tr_v)
            pltpu.sync_copy(rept_hbm.at[bb, :, pl.ds(l0, per_w)], rep_v)
            pltpu.sync_copy(idx_hbm.at[pl.ds(base, per_w)], idx_v)
            lanes = lax.iota(jnp.int32, _LANES)

            def group_body(g, carry):
                rows = g * _LANES + lanes
                idxv = plsc.load_gather(idx_v, [rows])
                acck = jnp.zeros((_LANES,), jnp.float32)
                accv = jnp.zeros((_LANES,), jnp.float32)
                for cc in range(c):
                    col = jnp.full((_LANES,), cc, jnp.int32)
                    trc = plsc.load_gather(tr_v, [col, rows])
                    kc = plsc.load_gather(keys_v, [col, idxv])
                    dk = trc - kc
                    acck = acck + dk * dk
                    rc = plsc.load_gather(rep_v, [col, rows])
                    vc = plsc.load_gather(values_v, [col, idxv])
                    dv = rc - vc
                    accv = accv + dv * dv
                plsc.store_scatter(outk_v, [rows], acck)
                plsc.store_scatter(outv_v, [rows], accv)
                return carry

            lax.fori_loop(0, n_groups, group_body, 0)
            pltpu.sync_copy(outk_v, outk_hbm.at[pl.ds(base, per_w)])
            pltpu.sync_copy(outv_v, outv_hbm.at[pl.ds(base, per_w)])

        return sc_mse

    return make


def kernel(trend_representation, representation, keys, values):
    b, l, c = trend_representation.shape
    m = keys.shape[0]
    trt3 = jnp.transpose(trend_representation, (0, 2, 1))
    rept3 = jnp.transpose(representation, (0, 2, 1))
    keys_t = keys.T
    values_t = values.T
    rows_per_chunk = b // _NCHUNKS
    chunk_t = rows_per_chunk * l
    sc_make = _make_sc_mse(b, l, c, m, chunk_t)
    kgs, vgs = [], []
    for i in range(_NCHUNKS):
        row_off = i * rows_per_chunk
        idx = _tc_argmax(trt3, keys_t, row_off, rows_per_chunk)
        kg, vg = sc_make(row_off)(trt3, rept3, keys_t, values_t, idx)
        kgs.append(kg)
        vgs.append(vg)
    kg = jnp.concatenate(kgs).reshape(b, l)
    vg = jnp.concatenate(vgs).reshape(b, l)
    return kg, vg
